# baseline (speedup 1.0000x reference)
import jax
import jax.numpy as jnp
from jax import lax
from jax.experimental import pallas as pl
from jax.experimental.pallas import tpu as pltpu
from jax.experimental.pallas import tpu_sc as plsc

N_STATES = 2097152
N_OBVS = 6
NC, NS, L = 2, 16, 16
NW = NC * NS
S_PER_W = N_STATES // NW
CH = 4096
N_CHUNKS = S_PER_W // CH
GROUPS = CH // L

_LN2 = 0.6931471805599453


def _ln(x):
    bits = lax.bitcast_convert_type(x, jnp.int32)
    e = ((bits >> 23) & 0xFF) - 127
    m = lax.bitcast_convert_type(
        (bits & 0x007FFFFF) | 0x3F800000, jnp.float32)
    big = m > 1.4142135623730951
    m = jnp.where(big, m * 0.5, m)
    e = e + jnp.where(big, 1, 0)
    z = (m - 1.0) / (m + 1.0)
    z2 = z * z
    p = z * (2.0 + z2 * (2.0 / 3.0 + z2 * (2.0 / 5.0 + z2 * (
        2.0 / 7.0 + z2 * (2.0 / 9.0)))))
    return e.astype(jnp.float32) * _LN2 + p


def _sc_body(state_hbm, obvs_hbm, probs_hbm, out_hbm,
             st_a, st_b, out_a, out_b, ov_v, pv_v,
             sin_a, sin_b, sout_a, sout_b):
    wid = lax.axis_index("s") * NC + lax.axis_index("c")
    w0 = wid * S_PER_W
    pltpu.async_copy(state_hbm.at[pl.ds(w0, CH)], st_a, sin_a)
    pltpu.async_copy(state_hbm.at[pl.ds(w0 + CH, CH)], st_b, sin_b)
    pltpu.sync_copy(obvs_hbm, ov_v.at[pl.ds(0, N_OBVS)])
    pltpu.sync_copy(probs_hbm, pv_v.at[pl.ds(0, 3)])
    iota = lax.iota(jnp.int32, L)
    pv = jnp.where(iota < 3, pv_v[...], 1.0)
    lpv = _ln(pv)
    ov = jnp.where(iota < N_OBVS, ov_v[...], 0)
    a = jnp.sum(jnp.where(iota == 0, lpv, 0.0))
    b = jnp.sum(jnp.where(iota == 1, lpv, 0.0))
    cc = jnp.sum(jnp.where(iota == 2, lpv, 0.0))
    avec = jnp.broadcast_to(a, (L,))
    nanv = jnp.full((L,), jnp.nan, jnp.float32)
    dsplats = []
    for j in range(N_OBVS):
        oj = jnp.sum(jnp.where(iota == j, ov, 0))
        dsplats.append(jnp.broadcast_to(jnp.where(oj == 6, b, cc), (L,)))

    def step(c, stb, oub, sin, sout):
        base = w0 + c * CH
        pltpu.make_async_copy(
            state_hbm.at[pl.ds(base, CH)], stb, sin).wait()

        @pl.when(c >= 2)
        def _():
            pltpu.make_async_copy(
                oub, out_hbm.at[:, pl.ds(base - 2 * CH, CH)], sout).wait()

        @plsc.parallel_loop(0, GROUPS)
        def body(g):
            s = stb[pl.ds(L * g, L)]
            m2 = s == 2
            t = jnp.where(s == 0, nanv, avec)
            for j in range(N_OBVS):
                oub[j, pl.ds(L * g, L)] = jnp.where(m2, dsplats[j], t)

        @pl.when(c + 2 < N_CHUNKS)
        def _():
            pltpu.async_copy(
                state_hbm.at[pl.ds(base + 2 * CH, CH)], stb, sin)

        pltpu.async_copy(oub, out_hbm.at[:, pl.ds(base, CH)], sout)

    def pair(t, carry):
        step(2 * t, st_a, out_a, sin_a, sout_a)
        step(2 * t + 1, st_b, out_b, sin_b, sout_b)
        return carry

    lax.fori_loop(0, N_CHUNKS // 2, pair, 0)
    pltpu.make_async_copy(
        out_a, out_hbm.at[:, pl.ds(w0 + (N_CHUNKS - 2) * CH, CH)],
        sout_a).wait()
    pltpu.make_async_copy(
        out_b, out_hbm.at[:, pl.ds(w0 + (N_CHUNKS - 1) * CH, CH)],
        sout_b).wait()


@jax.jit
def _expand(state, obvs, probs):
    mesh = plsc.VectorSubcoreMesh(core_axis_name="c", subcore_axis_name="s",
                                  num_cores=NC, num_subcores=NS)
    f = pl.kernel(
        _sc_body,
        out_type=jax.ShapeDtypeStruct((N_OBVS, N_STATES), jnp.float32),
        mesh=mesh,
        compiler_params=pltpu.CompilerParams(needs_layout_passes=False,
                                             use_tc_tiling_on_sc=True),
        scratch_types=[
            pltpu.VMEM((CH,), jnp.int32),
            pltpu.VMEM((CH,), jnp.int32),
            pltpu.VMEM((N_OBVS, CH), jnp.float32),
            pltpu.VMEM((N_OBVS, CH), jnp.float32),
            pltpu.VMEM((L,), jnp.int32),
            pltpu.VMEM((L,), jnp.float32),
            pltpu.SemaphoreType.DMA,
            pltpu.SemaphoreType.DMA,
            pltpu.SemaphoreType.DMA,
            pltpu.SemaphoreType.DMA,
        ],
    )
    return f(state, obvs, probs)


def kernel(state, obvs, probs):
    return _expand(state, obvs, probs).T
